# baseline (device time: 50331 ns/iter reference)
import jax
import jax.numpy as jnp
from jax import lax
from jax.experimental import pallas as pl
from jax.experimental.pallas import tpu as pltpu

Z = 4
M = 1024
MS = 256
D = 1024

_MESH = pl.DeviceIdType.MESH
_HBM = pltpu.MemorySpace.HBM


def kernel(partial, gamma):

    def body(x_hbm, g_hbm, out_hbm,
             xloc, g_ref, acc_ref, snd_ref, bx_ref, by_ref, bd_ref,
             ost, rr_buf, lr_buf,
             load_sems, g_sem, ost_sems,
             rr_send, rr_recv, lr_send, lr_recv,
             pb_send, pb_recv):
        my_x = lax.axis_index("x")
        my_y = lax.axis_index("y")
        my_z = lax.axis_index("z")
        r = 2 * my_x + my_y
        zp = jnp.minimum(my_z + 1, Z - 1)
        zm = jnp.maximum(my_z - 1, 0)

        def ldma(c):
            return pltpu.make_async_copy(
                x_hbm.at[0, pl.ds(c * M + r * MS, MS), :],
                xloc.at[c], load_sems.at[c])

        for c in (3, 0, 2, 1):
            ldma(c).start()
        gdma = pltpu.make_async_copy(g_hbm, g_ref, g_sem)
        gdma.start()

        def odma(q, src, i):
            return pltpu.make_async_copy(
                src, out_hbm.at[pl.ds(q * MS, MS), :], ost_sems.at[i])

        bsem = pltpu.get_barrier_semaphore()
        pl.semaphore_signal(bsem, inc=1, device_id=(1 - my_x, my_y, my_z),
                            device_id_type=_MESH)
        pl.semaphore_signal(bsem, inc=1, device_id=(my_x, 1 - my_y, my_z),
                            device_id_type=_MESH)

        @pl.when(my_z > 0)
        def _():
            pl.semaphore_signal(bsem, inc=1, device_id=(my_x, my_y, zm),
                                device_id_type=_MESH)

        @pl.when(my_z < Z - 1)
        def _():
            pl.semaphore_signal(bsem, inc=1, device_id=(my_x, my_y, zp),
                                device_id_type=_MESH)

        pl.semaphore_wait(bsem, 3)

        @pl.when((my_z > 0) & (my_z < Z - 1))
        def _():
            pl.semaphore_wait(bsem, 1)

        def rr_desc(c):
            return pltpu.make_async_remote_copy(
                src_ref=rr_buf.at[c], dst_ref=rr_buf.at[c],
                send_sem=rr_send.at[c], recv_sem=rr_recv.at[c],
                device_id=(my_x, my_y, zp), device_id_type=_MESH)

        def lr_desc(c):
            return pltpu.make_async_remote_copy(
                src_ref=lr_buf.at[c], dst_ref=lr_buf.at[c],
                send_sem=lr_send.at[c], recv_sem=lr_recv.at[c],
                device_id=(my_x, my_y, zm), device_id_type=_MESH)

        def rw_step(c):
            @pl.when(my_z == 0)
            def _():
                ldma(c).wait()
                rr_buf[c] = xloc[c].astype(jnp.bfloat16)
                rr_desc(c).start()

            @pl.when((my_z >= 1) & (my_z < c))
            def _():
                rr_desc(c).wait_recv()
                ldma(c).wait()
                rr_buf[c] = rr_buf[c] + xloc[c].astype(jnp.bfloat16)
                rr_desc(c).start()

        def lw_step(c):
            @pl.when(my_z == Z - 1)
            def _():
                ldma(c).wait()
                lr_buf[c] = xloc[c].astype(jnp.bfloat16)
                lr_desc(c).start()

            @pl.when((my_z <= Z - 2) & (my_z > c))
            def _():
                lr_desc(c).wait_recv()
                ldma(c).wait()
                lr_buf[c] = lr_buf[c] + xloc[c].astype(jnp.bfloat16)
                lr_desc(c).start()

        @pl.when(my_z % 2 == 1)
        def _():
            rw_step(3); lw_step(0); rw_step(2); lw_step(1)
            rw_step(1); lw_step(2)

        @pl.when(my_z % 2 == 0)
        def _():
            lw_step(0); rw_step(3); lw_step(1); rw_step(2)
            lw_step(2); rw_step(1)

        for c in range(Z):
            if c >= 1:
                @pl.when(my_z == c)
                def _(c=c):
                    rr_desc(c).wait_recv()
            if c <= Z - 2:
                @pl.when(my_z == c)
                def _(c=c):
                    lr_desc(c).wait_recv()

            @pl.when(my_z == c)
            def _(c=c):
                ldma(c).wait()
                acc = xloc[c]
                if c >= 1:
                    acc = acc + rr_buf[c].astype(jnp.float32)
                if c <= Z - 2:
                    acc = acc + lr_buf[c].astype(jnp.float32)
                acc_ref[...] = acc

        gdma.wait()
        y = acc_ref[...]
        rms = jnp.sqrt(jnp.mean(y * y, axis=-1, keepdims=True) + 1e-6)
        normed = y / rms * g_ref[...]
        acc_ref[...] = normed
        snd_ref[...] = normed.astype(jnp.bfloat16)
        odma(r, acc_ref, 0).start()

        s1x = pltpu.make_async_remote_copy(
            src_ref=snd_ref, dst_ref=bx_ref,
            send_sem=pb_send.at[0], recv_sem=pb_recv.at[0],
            device_id=(1 - my_x, my_y, my_z), device_id_type=_MESH)
        s1y = pltpu.make_async_remote_copy(
            src_ref=snd_ref, dst_ref=by_ref,
            send_sem=pb_send.at[1], recv_sem=pb_recv.at[1],
            device_id=(my_x, 1 - my_y, my_z), device_id_type=_MESH)
        s2 = pltpu.make_async_remote_copy(
            src_ref=bx_ref, dst_ref=bd_ref,
            send_sem=pb_send.at[2], recv_sem=pb_recv.at[2],
            device_id=(my_x, 1 - my_y, my_z), device_id_type=_MESH)
        s1x.start()
        s1y.start()
        r_x = 2 * (1 - my_x) + my_y
        r_y = 2 * my_x + (1 - my_y)
        r_d = 2 * (1 - my_x) + (1 - my_y)
        s1x.wait_recv()
        s2.start()
        ost[0] = bx_ref[...].astype(jnp.float32)
        odma(r_x, ost.at[0], 1).start()
        s1y.wait_recv()
        ost[1] = by_ref[...].astype(jnp.float32)
        odma(r_y, ost.at[1], 2).start()
        s2.wait_recv()
        ost[2] = bd_ref[...].astype(jnp.float32)
        odma(r_d, ost.at[2], 3).start()

        odma(r, acc_ref, 0).wait()
        odma(r_x, ost.at[0], 1).wait()
        odma(r_y, ost.at[1], 2).wait()
        odma(r_d, ost.at[2], 3).wait()
        s1x.wait_send()
        s1y.wait_send()
        s2.wait_send()
        for c in range(1, Z):
            @pl.when(my_z < c)
            def _(c=c):
                rr_desc(c).wait_send()
        for c in range(Z - 1):
            @pl.when(my_z > c)
            def _(c=c):
                lr_desc(c).wait_send()

    return pl.pallas_call(
        body,
        out_shape=jax.ShapeDtypeStruct((M, D), jnp.float32),
        in_specs=[
            pl.BlockSpec(memory_space=_HBM),
            pl.BlockSpec(memory_space=_HBM),
        ],
        out_specs=pl.BlockSpec(memory_space=_HBM),
        scratch_shapes=[
            pltpu.VMEM((Z, MS, D), jnp.float32),
            pltpu.VMEM((1, D), jnp.float32),
            pltpu.VMEM((MS, D), jnp.float32),
            pltpu.VMEM((MS, D), jnp.bfloat16),
            pltpu.VMEM((MS, D), jnp.bfloat16),
            pltpu.VMEM((MS, D), jnp.bfloat16),
            pltpu.VMEM((MS, D), jnp.bfloat16),
            pltpu.VMEM((3, MS, D), jnp.float32),
            pltpu.VMEM((Z, MS, D), jnp.bfloat16),
            pltpu.VMEM((Z, MS, D), jnp.bfloat16),
            pltpu.SemaphoreType.DMA((Z,)),
            pltpu.SemaphoreType.DMA,
            pltpu.SemaphoreType.DMA((Z,)),
            pltpu.SemaphoreType.DMA((Z,)),
            pltpu.SemaphoreType.DMA((Z,)),
            pltpu.SemaphoreType.DMA((Z,)),
            pltpu.SemaphoreType.DMA((Z,)),
            pltpu.SemaphoreType.DMA((3,)),
            pltpu.SemaphoreType.DMA((3,)),
        ],
        compiler_params=pltpu.CompilerParams(collective_id=0),
    )(partial, gamma.reshape(1, D))


# device time: 43305 ns/iter; 1.1622x vs baseline; 1.1622x over previous
import jax
import jax.numpy as jnp
from jax import lax
from jax.experimental import pallas as pl
from jax.experimental.pallas import tpu as pltpu

Z = 4
M = 1024
MS = 256
D = 1024

_MESH = pl.DeviceIdType.MESH


def kernel(partial, gamma):
    r_out = 2 * lax.axis_index("x") + lax.axis_index("y")
    xg = lax.dynamic_slice_in_dim(
        partial.reshape(Z, Z, MS, D), r_out, 1, axis=1
    ).reshape(Z, MS, D)

    def body(x_ref, g_ref, out_ref,
             acc_ref, snd_ref, bx_ref, by_ref, bd_ref,
             rr_buf, lr_buf,
             rr_send, rr_recv, lr_send, lr_recv,
             pb_send, pb_recv):
        my_x = lax.axis_index("x")
        my_y = lax.axis_index("y")
        my_z = lax.axis_index("z")
        r = 2 * my_x + my_y
        zp = jnp.minimum(my_z + 1, Z - 1)
        zm = jnp.maximum(my_z - 1, 0)

        bsem = pltpu.get_barrier_semaphore()
        pl.semaphore_signal(bsem, inc=1, device_id=(1 - my_x, my_y, my_z),
                            device_id_type=_MESH)
        pl.semaphore_signal(bsem, inc=1, device_id=(my_x, 1 - my_y, my_z),
                            device_id_type=_MESH)

        @pl.when(my_z > 0)
        def _():
            pl.semaphore_signal(bsem, inc=1, device_id=(my_x, my_y, zm),
                                device_id_type=_MESH)

        @pl.when(my_z < Z - 1)
        def _():
            pl.semaphore_signal(bsem, inc=1, device_id=(my_x, my_y, zp),
                                device_id_type=_MESH)

        pl.semaphore_wait(bsem, 3)

        @pl.when((my_z > 0) & (my_z < Z - 1))
        def _():
            pl.semaphore_wait(bsem, 1)

        def rr_desc(c):
            return pltpu.make_async_remote_copy(
                src_ref=rr_buf.at[c], dst_ref=rr_buf.at[c],
                send_sem=rr_send.at[c], recv_sem=rr_recv.at[c],
                device_id=(my_x, my_y, zp), device_id_type=_MESH)

        def lr_desc(c):
            return pltpu.make_async_remote_copy(
                src_ref=lr_buf.at[c], dst_ref=lr_buf.at[c],
                send_sem=lr_send.at[c], recv_sem=lr_recv.at[c],
                device_id=(my_x, my_y, zm), device_id_type=_MESH)

        def rw_step(c):
            @pl.when(my_z == 0)
            def _():
                rr_buf[c] = x_ref[c].astype(jnp.bfloat16)
                rr_desc(c).start()

            @pl.when((my_z >= 1) & (my_z < c))
            def _():
                rr_desc(c).wait_recv()
                rr_buf[c] = rr_buf[c] + x_ref[c].astype(jnp.bfloat16)
                rr_desc(c).start()

        def lw_step(c):
            @pl.when(my_z == Z - 1)
            def _():
                lr_buf[c] = x_ref[c].astype(jnp.bfloat16)
                lr_desc(c).start()

            @pl.when((my_z <= Z - 2) & (my_z > c))
            def _():
                lr_desc(c).wait_recv()
                lr_buf[c] = lr_buf[c] + x_ref[c].astype(jnp.bfloat16)
                lr_desc(c).start()

        @pl.when(my_z % 2 == 1)
        def _():
            rw_step(3); lw_step(0); rw_step(2); lw_step(1)
            rw_step(1); lw_step(2)

        @pl.when(my_z % 2 == 0)
        def _():
            lw_step(0); rw_step(3); lw_step(1); rw_step(2)
            lw_step(2); rw_step(1)

        for c in range(Z):
            if c >= 1:
                @pl.when(my_z == c)
                def _(c=c):
                    rr_desc(c).wait_recv()
            if c <= Z - 2:
                @pl.when(my_z == c)
                def _(c=c):
                    lr_desc(c).wait_recv()

            @pl.when(my_z == c)
            def _(c=c):
                acc = x_ref[c]
                if c >= 1:
                    acc = acc + rr_buf[c].astype(jnp.float32)
                if c <= Z - 2:
                    acc = acc + lr_buf[c].astype(jnp.float32)
                acc_ref[...] = acc

        y = acc_ref[...]
        rms = jnp.sqrt(jnp.mean(y * y, axis=-1, keepdims=True) + 1e-6)
        normed = (y / rms * g_ref[...]).astype(jnp.bfloat16)
        snd_ref[...] = normed
        out_ref[pl.ds(r * MS, MS), :] = normed

        s1x = pltpu.make_async_remote_copy(
            src_ref=snd_ref, dst_ref=bx_ref,
            send_sem=pb_send.at[0], recv_sem=pb_recv.at[0],
            device_id=(1 - my_x, my_y, my_z), device_id_type=_MESH)
        s1y = pltpu.make_async_remote_copy(
            src_ref=snd_ref, dst_ref=by_ref,
            send_sem=pb_send.at[1], recv_sem=pb_recv.at[1],
            device_id=(my_x, 1 - my_y, my_z), device_id_type=_MESH)
        s2 = pltpu.make_async_remote_copy(
            src_ref=bx_ref, dst_ref=bd_ref,
            send_sem=pb_send.at[2], recv_sem=pb_recv.at[2],
            device_id=(my_x, 1 - my_y, my_z), device_id_type=_MESH)
        s1x.start()
        s1y.start()
        r_x = 2 * (1 - my_x) + my_y
        r_y = 2 * my_x + (1 - my_y)
        r_d = 2 * (1 - my_x) + (1 - my_y)
        s1x.wait_recv()
        s2.start()
        out_ref[pl.ds(r_x * MS, MS), :] = bx_ref[...]
        s1y.wait_recv()
        out_ref[pl.ds(r_y * MS, MS), :] = by_ref[...]
        s2.wait_recv()
        out_ref[pl.ds(r_d * MS, MS), :] = bd_ref[...]

        s1x.wait_send()
        s1y.wait_send()
        s2.wait_send()
        for c in range(1, Z):
            @pl.when(my_z < c)
            def _(c=c):
                rr_desc(c).wait_send()
        for c in range(Z - 1):
            @pl.when(my_z > c)
            def _(c=c):
                lr_desc(c).wait_send()

    return pl.pallas_call(
        body,
        out_shape=jax.ShapeDtypeStruct((M, D), jnp.bfloat16),
        in_specs=[
            pl.BlockSpec(memory_space=pltpu.VMEM),
            pl.BlockSpec(memory_space=pltpu.VMEM),
        ],
        out_specs=pl.BlockSpec(memory_space=pltpu.VMEM),
        scratch_shapes=[
            pltpu.VMEM((MS, D), jnp.float32),
            pltpu.VMEM((MS, D), jnp.bfloat16),
            pltpu.VMEM((MS, D), jnp.bfloat16),
            pltpu.VMEM((MS, D), jnp.bfloat16),
            pltpu.VMEM((MS, D), jnp.bfloat16),
            pltpu.VMEM((Z, MS, D), jnp.bfloat16),
            pltpu.VMEM((Z, MS, D), jnp.bfloat16),
            pltpu.SemaphoreType.DMA((Z,)),
            pltpu.SemaphoreType.DMA((Z,)),
            pltpu.SemaphoreType.DMA((Z,)),
            pltpu.SemaphoreType.DMA((Z,)),
            pltpu.SemaphoreType.DMA((3,)),
            pltpu.SemaphoreType.DMA((3,)),
        ],
        compiler_params=pltpu.CompilerParams(collective_id=0),
    )(xg, gamma.reshape(1, D))


# device time: 40566 ns/iter; 1.2407x vs baseline; 1.0675x over previous
import jax
import jax.numpy as jnp
from jax import lax
from jax.experimental import pallas as pl
from jax.experimental.pallas import tpu as pltpu

Z = 4
M = 1024
MS = 256
D = 1024

_MESH = pl.DeviceIdType.MESH


def kernel(partial, gamma):
    r_out = 2 * lax.axis_index("x") + lax.axis_index("y")
    xg = lax.dynamic_slice_in_dim(
        partial.reshape(Z, Z, MS, D), r_out, 1, axis=1
    ).reshape(Z, MS, D)

    def body(x_ref, g_ref, out_ref,
             acc_ref, snd_ref, bx_ref, by_ref, bd_ref,
             rr_buf, lr_buf,
             rr_send, rr_recv, lr_send, lr_recv,
             pb_send, pb_recv):
        my_x = lax.axis_index("x")
        my_y = lax.axis_index("y")
        my_z = lax.axis_index("z")
        r = 2 * my_x + my_y
        zp = jnp.minimum(my_z + 1, Z - 1)
        zm = jnp.maximum(my_z - 1, 0)

        bsem = pltpu.get_barrier_semaphore()
        pl.semaphore_signal(bsem, inc=1, device_id=(1 - my_x, my_y, my_z),
                            device_id_type=_MESH)
        pl.semaphore_signal(bsem, inc=1, device_id=(my_x, 1 - my_y, my_z),
                            device_id_type=_MESH)

        @pl.when(my_z > 0)
        def _():
            pl.semaphore_signal(bsem, inc=1, device_id=(my_x, my_y, zm),
                                device_id_type=_MESH)

        @pl.when(my_z < Z - 1)
        def _():
            pl.semaphore_signal(bsem, inc=1, device_id=(my_x, my_y, zp),
                                device_id_type=_MESH)

        pl.semaphore_wait(bsem, 3)

        @pl.when((my_z > 0) & (my_z < Z - 1))
        def _():
            pl.semaphore_wait(bsem, 1)

        def rr_desc(c):
            return pltpu.make_async_remote_copy(
                src_ref=rr_buf.at[c], dst_ref=rr_buf.at[c],
                send_sem=rr_send.at[c], recv_sem=rr_recv.at[c],
                device_id=(my_x, my_y, zp), device_id_type=_MESH)

        def lr_desc(c):
            return pltpu.make_async_remote_copy(
                src_ref=lr_buf.at[c], dst_ref=lr_buf.at[c],
                send_sem=lr_send.at[c], recv_sem=lr_recv.at[c],
                device_id=(my_x, my_y, zm), device_id_type=_MESH)

        def rw_step(c):
            @pl.when(my_z == 0)
            def _():
                rr_buf[c] = x_ref[c].astype(jnp.bfloat16)
                rr_desc(c).start()

            @pl.when((my_z >= 1) & (my_z < c))
            def _():
                rr_desc(c).wait_recv()
                rr_buf[c] = rr_buf[c] + x_ref[c].astype(jnp.bfloat16)
                rr_desc(c).start()

        def lw_step(c):
            @pl.when(my_z == Z - 1)
            def _():
                lr_buf[c] = x_ref[c].astype(jnp.bfloat16)
                lr_desc(c).start()

            @pl.when((my_z <= Z - 2) & (my_z > c))
            def _():
                lr_desc(c).wait_recv()
                lr_buf[c] = lr_buf[c] + x_ref[c].astype(jnp.bfloat16)
                lr_desc(c).start()

        @pl.when(my_z % 2 == 1)
        def _():
            rw_step(3); lw_step(0); rw_step(2); lw_step(1)
            rw_step(1); lw_step(2)

        @pl.when(my_z % 2 == 0)
        def _():
            lw_step(0); rw_step(3); lw_step(1); rw_step(2)
            lw_step(2); rw_step(1)

        for c in range(Z):
            if c >= 1:
                @pl.when(my_z == c)
                def _(c=c):
                    rr_desc(c).wait_recv()
            if c <= Z - 2:
                @pl.when(my_z == c)
                def _(c=c):
                    lr_desc(c).wait_recv()

            @pl.when(my_z == c)
            def _(c=c):
                acc = x_ref[c]
                if c >= 1:
                    acc = acc + rr_buf[c].astype(jnp.float32)
                if c <= Z - 2:
                    acc = acc + lr_buf[c].astype(jnp.float32)
                acc_ref[...] = acc

        y = acc_ref[...]
        rms = jnp.sqrt(jnp.mean(y * y, axis=-1, keepdims=True) + 1e-6)
        normed = (y / rms * g_ref[...]).astype(jnp.bfloat16)
        snd_ref[...] = normed
        out_ref[pl.ds(r * MS, MS), :] = normed

        H = MS // 2
        s1x = pltpu.make_async_remote_copy(
            src_ref=snd_ref, dst_ref=bx_ref,
            send_sem=pb_send.at[0], recv_sem=pb_recv.at[0],
            device_id=(1 - my_x, my_y, my_z), device_id_type=_MESH)
        s1y = pltpu.make_async_remote_copy(
            src_ref=snd_ref, dst_ref=by_ref,
            send_sem=pb_send.at[1], recv_sem=pb_recv.at[1],
            device_id=(my_x, 1 - my_y, my_z), device_id_type=_MESH)
        s2x = pltpu.make_async_remote_copy(
            src_ref=by_ref.at[pl.ds(0, H), :],
            dst_ref=bd_ref.at[pl.ds(0, H), :],
            send_sem=pb_send.at[2], recv_sem=pb_recv.at[2],
            device_id=(1 - my_x, my_y, my_z), device_id_type=_MESH)
        s2y = pltpu.make_async_remote_copy(
            src_ref=bx_ref.at[pl.ds(H, H), :],
            dst_ref=bd_ref.at[pl.ds(H, H), :],
            send_sem=pb_send.at[3], recv_sem=pb_recv.at[3],
            device_id=(my_x, 1 - my_y, my_z), device_id_type=_MESH)
        s1x.start()
        s1y.start()
        r_x = 2 * (1 - my_x) + my_y
        r_y = 2 * my_x + (1 - my_y)
        r_d = 2 * (1 - my_x) + (1 - my_y)
        s1x.wait_recv()
        s2y.start()
        out_ref[pl.ds(r_x * MS, MS), :] = bx_ref[...]
        s1y.wait_recv()
        s2x.start()
        out_ref[pl.ds(r_y * MS, MS), :] = by_ref[...]
        s2x.wait_recv()
        s2y.wait_recv()
        out_ref[pl.ds(r_d * MS, MS), :] = bd_ref[...]

        s1x.wait_send()
        s1y.wait_send()
        s2x.wait_send()
        s2y.wait_send()
        for c in range(1, Z):
            @pl.when(my_z < c)
            def _(c=c):
                rr_desc(c).wait_send()
        for c in range(Z - 1):
            @pl.when(my_z > c)
            def _(c=c):
                lr_desc(c).wait_send()

    return pl.pallas_call(
        body,
        out_shape=jax.ShapeDtypeStruct((M, D), jnp.bfloat16),
        in_specs=[
            pl.BlockSpec(memory_space=pltpu.VMEM),
            pl.BlockSpec(memory_space=pltpu.VMEM),
        ],
        out_specs=pl.BlockSpec(memory_space=pltpu.VMEM),
        scratch_shapes=[
            pltpu.VMEM((MS, D), jnp.float32),
            pltpu.VMEM((MS, D), jnp.bfloat16),
            pltpu.VMEM((MS, D), jnp.bfloat16),
            pltpu.VMEM((MS, D), jnp.bfloat16),
            pltpu.VMEM((MS, D), jnp.bfloat16),
            pltpu.VMEM((Z, MS, D), jnp.bfloat16),
            pltpu.VMEM((Z, MS, D), jnp.bfloat16),
            pltpu.SemaphoreType.DMA((Z,)),
            pltpu.SemaphoreType.DMA((Z,)),
            pltpu.SemaphoreType.DMA((Z,)),
            pltpu.SemaphoreType.DMA((Z,)),
            pltpu.SemaphoreType.DMA((4,)),
            pltpu.SemaphoreType.DMA((4,)),
        ],
        compiler_params=pltpu.CompilerParams(collective_id=0),
    )(xg, gamma.reshape(1, D))


# device time: 37157 ns/iter; 1.3545x vs baseline; 1.0917x over previous
import jax
import jax.numpy as jnp
from jax import lax
from jax.experimental import pallas as pl
from jax.experimental.pallas import tpu as pltpu

Z = 4
M = 1024
MS = 256
H = MS // 2
HQ = H // 2
D = 1024

_MESH = pl.DeviceIdType.MESH


def kernel(partial, gamma):
    r_out = 2 * lax.axis_index("x") + lax.axis_index("y")
    xg = lax.dynamic_slice_in_dim(
        partial.reshape(Z, Z, MS, D), r_out, 1, axis=1
    ).reshape(Z, MS, D)

    def body(x_ref, g_ref, out_ref,
             snd_ref, bx_ref, by_ref, bd_ref,
             rr_buf, lr_buf,
             rr_send, rr_recv, lr_send, lr_recv,
             pb_send, pb_recv):
        my_x = lax.axis_index("x")
        my_y = lax.axis_index("y")
        my_z = lax.axis_index("z")
        r = 2 * my_x + my_y
        zp = jnp.minimum(my_z + 1, Z - 1)
        zm = jnp.maximum(my_z - 1, 0)

        bsem = pltpu.get_barrier_semaphore()
        pl.semaphore_signal(bsem, inc=1, device_id=(1 - my_x, my_y, my_z),
                            device_id_type=_MESH)
        pl.semaphore_signal(bsem, inc=1, device_id=(my_x, 1 - my_y, my_z),
                            device_id_type=_MESH)

        @pl.when(my_z > 0)
        def _():
            pl.semaphore_signal(bsem, inc=1, device_id=(my_x, my_y, zm),
                                device_id_type=_MESH)

        @pl.when(my_z < Z - 1)
        def _():
            pl.semaphore_signal(bsem, inc=1, device_id=(my_x, my_y, zp),
                                device_id_type=_MESH)

        pl.semaphore_wait(bsem, 3)

        @pl.when((my_z > 0) & (my_z < Z - 1))
        def _():
            pl.semaphore_wait(bsem, 1)

        def rr_desc(c, h):
            s = 2 * c + h
            return pltpu.make_async_remote_copy(
                src_ref=rr_buf.at[s], dst_ref=rr_buf.at[s],
                send_sem=rr_send.at[s], recv_sem=rr_recv.at[s],
                device_id=(my_x, my_y, zp), device_id_type=_MESH)

        def lr_desc(c, h):
            s = 2 * c + h
            return pltpu.make_async_remote_copy(
                src_ref=lr_buf.at[s], dst_ref=lr_buf.at[s],
                send_sem=lr_send.at[s], recv_sem=lr_recv.at[s],
                device_id=(my_x, my_y, zm), device_id_type=_MESH)

        def xh(c, h):
            return x_ref[c, pl.ds(h * H, H), :]

        def rw_edge(c, h):
            @pl.when(my_z == 0)
            def _():
                rr_buf[2 * c + h] = xh(c, h).astype(jnp.bfloat16)
                rr_desc(c, h).start()

        def rw_mid(c, h):
            @pl.when((my_z >= 1) & (my_z < c))
            def _():
                rr_desc(c, h).wait_recv()
                rr_buf[2 * c + h] = (
                    rr_buf[2 * c + h] + xh(c, h).astype(jnp.bfloat16))
                rr_desc(c, h).start()

        def lw_edge(c, h):
            @pl.when(my_z == Z - 1)
            def _():
                lr_buf[2 * c + h] = xh(c, h).astype(jnp.bfloat16)
                lr_desc(c, h).start()

        def lw_mid(c, h):
            @pl.when((my_z <= Z - 2) & (my_z > c))
            def _():
                lr_desc(c, h).wait_recv()
                lr_buf[2 * c + h] = (
                    lr_buf[2 * c + h] + xh(c, h).astype(jnp.bfloat16))
                lr_desc(c, h).start()

        def rw_step(c, h):
            rw_edge(c, h)
            rw_mid(c, h)

        def lw_step(c, h):
            lw_edge(c, h)
            lw_mid(c, h)

        @pl.when(my_z % 2 == 1)
        def _():
            rw_step(3, 0); lw_step(0, 0); rw_step(2, 0); lw_step(1, 0)
            rw_step(1, 0); lw_step(2, 0)

        @pl.when(my_z % 2 == 0)
        def _():
            lw_step(0, 0); rw_step(3, 0); lw_step(1, 0); rw_step(2, 0)
            lw_step(2, 0); rw_step(1, 0)

        for c in (3, 2, 1):
            rw_edge(c, 1)
        for c in (0, 1, 2):
            lw_edge(c, 1)

        def pb_desc(i, h):
            s = 2 * i + h
            if i == 0:
                return pltpu.make_async_remote_copy(
                    src_ref=snd_ref.at[pl.ds(h * H, H), :],
                    dst_ref=bx_ref.at[pl.ds(h * H, H), :],
                    send_sem=pb_send.at[s], recv_sem=pb_recv.at[s],
                    device_id=(1 - my_x, my_y, my_z), device_id_type=_MESH)
            if i == 1:
                return pltpu.make_async_remote_copy(
                    src_ref=snd_ref.at[pl.ds(h * H, H), :],
                    dst_ref=by_ref.at[pl.ds(h * H, H), :],
                    send_sem=pb_send.at[s], recv_sem=pb_recv.at[s],
                    device_id=(my_x, 1 - my_y, my_z), device_id_type=_MESH)
            if i == 2:
                return pltpu.make_async_remote_copy(
                    src_ref=by_ref.at[pl.ds(h * H, HQ), :],
                    dst_ref=bd_ref.at[pl.ds(h * H, HQ), :],
                    send_sem=pb_send.at[s], recv_sem=pb_recv.at[s],
                    device_id=(1 - my_x, my_y, my_z), device_id_type=_MESH)
            return pltpu.make_async_remote_copy(
                src_ref=bx_ref.at[pl.ds(h * H + HQ, HQ), :],
                dst_ref=bd_ref.at[pl.ds(h * H + HQ, HQ), :],
                send_sem=pb_send.at[s], recv_sem=pb_recv.at[s],
                device_id=(my_x, 1 - my_y, my_z), device_id_type=_MESH)

        r_x = 2 * (1 - my_x) + my_y
        r_y = 2 * my_x + (1 - my_y)
        r_d = 2 * (1 - my_x) + (1 - my_y)

        def combine_norm_bstart(h):
            for c in range(Z):
                if c >= 1:
                    @pl.when(my_z == c)
                    def _(c=c):
                        rr_desc(c, h).wait_recv()
                if c <= Z - 2:
                    @pl.when(my_z == c)
                    def _(c=c):
                        lr_desc(c, h).wait_recv()

                @pl.when(my_z == c)
                def _(c=c):
                    acc = xh(c, h)
                    if c >= 1:
                        acc = acc + rr_buf[2 * c + h].astype(jnp.float32)
                    if c <= Z - 2:
                        acc = acc + lr_buf[2 * c + h].astype(jnp.float32)
                    rms = jnp.sqrt(
                        jnp.mean(acc * acc, axis=-1, keepdims=True) + 1e-6)
                    snd_ref[pl.ds(h * H, H), :] = (
                        acc / rms * g_ref[...]).astype(jnp.bfloat16)

            out_ref[pl.ds(r * MS + h * H, H), :] = snd_ref[pl.ds(h * H, H), :]
            pb_desc(0, h).start()
            pb_desc(1, h).start()

        def b_mid(h):
            pb_desc(0, h).wait_recv()
            pb_desc(3, h).start()
            out_ref[pl.ds(r_x * MS + h * H, H), :] = \
                bx_ref[pl.ds(h * H, H), :]
            pb_desc(1, h).wait_recv()
            pb_desc(2, h).start()
            out_ref[pl.ds(r_y * MS + h * H, H), :] = \
                by_ref[pl.ds(h * H, H), :]

        def b_fin(h):
            pb_desc(2, h).wait_recv()
            pb_desc(3, h).wait_recv()
            out_ref[pl.ds(r_d * MS + h * H, H), :] = \
                bd_ref[pl.ds(h * H, H), :]

        combine_norm_bstart(0)

        @pl.when(my_z % 2 == 1)
        def _():
            rw_mid(3, 1); lw_mid(0, 1); rw_mid(2, 1); lw_mid(1, 1)
            rw_mid(1, 1); lw_mid(2, 1)

        @pl.when(my_z % 2 == 0)
        def _():
            lw_mid(0, 1); rw_mid(3, 1); lw_mid(1, 1); rw_mid(2, 1)
            lw_mid(2, 1); rw_mid(1, 1)

        b_mid(0)
        combine_norm_bstart(1)
        b_fin(0)
        b_mid(1)
        b_fin(1)

        for h in (0, 1):
            for i in range(4):
                pb_desc(i, h).wait_send()
            for c in range(1, Z):
                @pl.when(my_z < c)
                def _(c=c, h=h):
                    rr_desc(c, h).wait_send()
            for c in range(Z - 1):
                @pl.when(my_z > c)
                def _(c=c, h=h):
                    lr_desc(c, h).wait_send()

    return pl.pallas_call(
        body,
        out_shape=jax.ShapeDtypeStruct((M, D), jnp.bfloat16),
        in_specs=[
            pl.BlockSpec(memory_space=pltpu.VMEM),
            pl.BlockSpec(memory_space=pltpu.VMEM),
        ],
        out_specs=pl.BlockSpec(memory_space=pltpu.VMEM),
        scratch_shapes=[
            pltpu.VMEM((MS, D), jnp.bfloat16),
            pltpu.VMEM((MS, D), jnp.bfloat16),
            pltpu.VMEM((MS, D), jnp.bfloat16),
            pltpu.VMEM((MS, D), jnp.bfloat16),
            pltpu.VMEM((2 * Z, H, D), jnp.bfloat16),
            pltpu.VMEM((2 * Z, H, D), jnp.bfloat16),
            pltpu.SemaphoreType.DMA((2 * Z,)),
            pltpu.SemaphoreType.DMA((2 * Z,)),
            pltpu.SemaphoreType.DMA((2 * Z,)),
            pltpu.SemaphoreType.DMA((2 * Z,)),
            pltpu.SemaphoreType.DMA((8,)),
            pltpu.SemaphoreType.DMA((8,)),
        ],
        compiler_params=pltpu.CompilerParams(collective_id=0),
    )(xg, gamma.reshape(1, D))


# device time: 37120 ns/iter; 1.3559x vs baseline; 1.0010x over previous
import jax
import jax.numpy as jnp
from jax import lax
from jax.experimental import pallas as pl
from jax.experimental.pallas import tpu as pltpu

Z = 4
M = 1024
MS = 256
H = MS // 2
HQ = H // 2
D = 1024

_MESH = pl.DeviceIdType.MESH


def kernel(partial, gamma):
    r_out = 2 * lax.axis_index("x") + lax.axis_index("y")
    xg = lax.dynamic_slice_in_dim(
        partial.reshape(Z, Z, MS, D), r_out, 1, axis=1
    ).reshape(Z, MS, D)

    def body(x_ref, g_ref, out_ref,
             rr_buf, lr_buf,
             rr_send, rr_recv, lr_send, lr_recv,
             pb_send, pb_recv):
        my_x = lax.axis_index("x")
        my_y = lax.axis_index("y")
        my_z = lax.axis_index("z")
        r = 2 * my_x + my_y
        zp = jnp.minimum(my_z + 1, Z - 1)
        zm = jnp.maximum(my_z - 1, 0)

        bsem = pltpu.get_barrier_semaphore()
        pl.semaphore_signal(bsem, inc=1, device_id=(1 - my_x, my_y, my_z),
                            device_id_type=_MESH)
        pl.semaphore_signal(bsem, inc=1, device_id=(my_x, 1 - my_y, my_z),
                            device_id_type=_MESH)

        @pl.when(my_z > 0)
        def _():
            pl.semaphore_signal(bsem, inc=1, device_id=(my_x, my_y, zm),
                                device_id_type=_MESH)

        @pl.when(my_z < Z - 1)
        def _():
            pl.semaphore_signal(bsem, inc=1, device_id=(my_x, my_y, zp),
                                device_id_type=_MESH)

        pl.semaphore_wait(bsem, 3)

        @pl.when((my_z > 0) & (my_z < Z - 1))
        def _():
            pl.semaphore_wait(bsem, 1)

        def rr_desc(c, h):
            s = 2 * c + h
            return pltpu.make_async_remote_copy(
                src_ref=rr_buf.at[s], dst_ref=rr_buf.at[s],
                send_sem=rr_send.at[s], recv_sem=rr_recv.at[s],
                device_id=(my_x, my_y, zp), device_id_type=_MESH)

        def lr_desc(c, h):
            s = 2 * c + h
            return pltpu.make_async_remote_copy(
                src_ref=lr_buf.at[s], dst_ref=lr_buf.at[s],
                send_sem=lr_send.at[s], recv_sem=lr_recv.at[s],
                device_id=(my_x, my_y, zm), device_id_type=_MESH)

        def xh(c, h):
            return x_ref[c, pl.ds(h * H, H), :]

        def rw_edge(c, h):
            @pl.when(my_z == 0)
            def _():
                rr_buf[2 * c + h] = xh(c, h).astype(jnp.bfloat16)
                rr_desc(c, h).start()

        def rw_mid(c, h):
            @pl.when((my_z >= 1) & (my_z < c))
            def _():
                rr_desc(c, h).wait_recv()
                rr_buf[2 * c + h] = (
                    rr_buf[2 * c + h] + xh(c, h).astype(jnp.bfloat16))
                rr_desc(c, h).start()

        def lw_edge(c, h):
            @pl.when(my_z == Z - 1)
            def _():
                lr_buf[2 * c + h] = xh(c, h).astype(jnp.bfloat16)
                lr_desc(c, h).start()

        def lw_mid(c, h):
            @pl.when((my_z <= Z - 2) & (my_z > c))
            def _():
                lr_desc(c, h).wait_recv()
                lr_buf[2 * c + h] = (
                    lr_buf[2 * c + h] + xh(c, h).astype(jnp.bfloat16))
                lr_desc(c, h).start()

        def rw_step(c, h):
            rw_edge(c, h)
            rw_mid(c, h)

        def lw_step(c, h):
            lw_edge(c, h)
            lw_mid(c, h)

        @pl.when(my_z % 2 == 1)
        def _():
            rw_step(3, 0); lw_step(0, 0); rw_step(2, 0); lw_step(1, 0)
            rw_step(1, 0); lw_step(2, 0)

        @pl.when(my_z % 2 == 0)
        def _():
            lw_step(0, 0); rw_step(3, 0); lw_step(1, 0); rw_step(2, 0)
            lw_step(2, 0); rw_step(1, 0)

        for c in (3, 2, 1):
            rw_edge(c, 1)
        for c in (0, 1, 2):
            lw_edge(c, 1)

        r_x = 2 * (1 - my_x) + my_y
        r_y = 2 * my_x + (1 - my_y)

        def pb_desc(i, h):
            s = 2 * i + h
            if i == 0:
                sl = out_ref.at[pl.ds(r * MS + h * H, H), :]
                return pltpu.make_async_remote_copy(
                    src_ref=sl, dst_ref=sl,
                    send_sem=pb_send.at[s], recv_sem=pb_recv.at[s],
                    device_id=(1 - my_x, my_y, my_z), device_id_type=_MESH)
            if i == 1:
                sl = out_ref.at[pl.ds(r * MS + h * H, H), :]
                return pltpu.make_async_remote_copy(
                    src_ref=sl, dst_ref=sl,
                    send_sem=pb_send.at[s], recv_sem=pb_recv.at[s],
                    device_id=(my_x, 1 - my_y, my_z), device_id_type=_MESH)
            if i == 2:
                sl = out_ref.at[pl.ds(r_y * MS + h * H, HQ), :]
                return pltpu.make_async_remote_copy(
                    src_ref=sl, dst_ref=sl,
                    send_sem=pb_send.at[s], recv_sem=pb_recv.at[s],
                    device_id=(1 - my_x, my_y, my_z), device_id_type=_MESH)
            sl = out_ref.at[pl.ds(r_x * MS + h * H + HQ, HQ), :]
            return pltpu.make_async_remote_copy(
                src_ref=sl, dst_ref=sl,
                send_sem=pb_send.at[s], recv_sem=pb_recv.at[s],
                device_id=(my_x, 1 - my_y, my_z), device_id_type=_MESH)

        def combine_norm_bstart(h):
            for c in range(Z):
                if c >= 1:
                    @pl.when(my_z == c)
                    def _(c=c):
                        rr_desc(c, h).wait_recv()
                if c <= Z - 2:
                    @pl.when(my_z == c)
                    def _(c=c):
                        lr_desc(c, h).wait_recv()

                @pl.when(my_z == c)
                def _(c=c):
                    acc = xh(c, h)
                    if c >= 1:
                        acc = acc + rr_buf[2 * c + h].astype(jnp.float32)
                    if c <= Z - 2:
                        acc = acc + lr_buf[2 * c + h].astype(jnp.float32)
                    inv = lax.rsqrt(
                        jnp.mean(acc * acc, axis=-1, keepdims=True) + 1e-6)
                    out_ref[pl.ds(r * MS + h * H, H), :] = (
                        acc * inv * g_ref[...]).astype(jnp.bfloat16)

            pb_desc(0, h).start()
            pb_desc(1, h).start()

        def b_mid(h):
            pb_desc(0, h).wait_recv()
            pb_desc(3, h).start()
            pb_desc(1, h).wait_recv()
            pb_desc(2, h).start()

        def b_fin(h):
            pb_desc(2, h).wait_recv()
            pb_desc(3, h).wait_recv()

        combine_norm_bstart(0)

        @pl.when(my_z % 2 == 1)
        def _():
            rw_mid(3, 1); lw_mid(0, 1); rw_mid(2, 1); lw_mid(1, 1)
            rw_mid(1, 1); lw_mid(2, 1)

        @pl.when(my_z % 2 == 0)
        def _():
            lw_mid(0, 1); rw_mid(3, 1); lw_mid(1, 1); rw_mid(2, 1)
            lw_mid(2, 1); rw_mid(1, 1)

        b_mid(0)
        combine_norm_bstart(1)
        b_fin(0)
        b_mid(1)
        b_fin(1)

        for h in (0, 1):
            for i in range(4):
                pb_desc(i, h).wait_send()
            for c in range(1, Z):
                @pl.when(my_z < c)
                def _(c=c, h=h):
                    rr_desc(c, h).wait_send()
            for c in range(Z - 1):
                @pl.when(my_z > c)
                def _(c=c, h=h):
                    lr_desc(c, h).wait_send()

    return pl.pallas_call(
        body,
        out_shape=jax.ShapeDtypeStruct((M, D), jnp.bfloat16),
        in_specs=[
            pl.BlockSpec(memory_space=pltpu.VMEM),
            pl.BlockSpec(memory_space=pltpu.VMEM),
        ],
        out_specs=pl.BlockSpec(memory_space=pltpu.VMEM),
        scratch_shapes=[
            pltpu.VMEM((2 * Z, H, D), jnp.bfloat16),
            pltpu.VMEM((2 * Z, H, D), jnp.bfloat16),
            pltpu.SemaphoreType.DMA((2 * Z,)),
            pltpu.SemaphoreType.DMA((2 * Z,)),
            pltpu.SemaphoreType.DMA((2 * Z,)),
            pltpu.SemaphoreType.DMA((2 * Z,)),
            pltpu.SemaphoreType.DMA((8,)),
            pltpu.SemaphoreType.DMA((8,)),
        ],
        compiler_params=pltpu.CompilerParams(collective_id=0),
    )(xg, gamma.reshape(1, D))


# device time: 32294 ns/iter; 1.5585x vs baseline; 1.1494x over previous
import jax
import jax.numpy as jnp
from jax import lax
from jax.experimental import pallas as pl
from jax.experimental.pallas import tpu as pltpu

Z = 4
M = 1024
MS = 256
H = MS // 2
HQ = H // 2
D = 1024

_MESH = pl.DeviceIdType.MESH


def kernel(partial, gamma):
    r_out = 2 * lax.axis_index("x") + lax.axis_index("y")
    xg = lax.dynamic_slice_in_dim(
        partial.reshape(Z, Z, MS, D), r_out, 1, axis=1
    ).reshape(Z, MS, D)

    def body(x_ref, g_ref, out_ref,
             rr_buf, lr_buf, rr2_buf, lr2_buf,
             rr_send, rr_recv, lr_send, lr_recv,
             rr2_send, rr2_recv, lr2_send, lr2_recv,
             pb_send, pb_recv):
        my_x = lax.axis_index("x")
        my_y = lax.axis_index("y")
        my_z = lax.axis_index("z")
        r = 2 * my_x + my_y
        zp = jnp.minimum(my_z + 1, Z - 1)
        zm = jnp.maximum(my_z - 1, 0)

        bsem = pltpu.get_barrier_semaphore()
        pl.semaphore_signal(bsem, inc=1, device_id=(1 - my_x, my_y, my_z),
                            device_id_type=_MESH)
        pl.semaphore_signal(bsem, inc=1, device_id=(my_x, 1 - my_y, my_z),
                            device_id_type=_MESH)

        @pl.when(my_z > 0)
        def _():
            pl.semaphore_signal(bsem, inc=1, device_id=(my_x, my_y, zm),
                                device_id_type=_MESH)

        @pl.when(my_z < Z - 1)
        def _():
            pl.semaphore_signal(bsem, inc=1, device_id=(my_x, my_y, zp),
                                device_id_type=_MESH)

        pl.semaphore_wait(bsem, 3)

        @pl.when((my_z > 0) & (my_z < Z - 1))
        def _():
            pl.semaphore_wait(bsem, 1)

        def rr_desc(c, h, tz=None):
            s = 2 * c + h
            return pltpu.make_async_remote_copy(
                src_ref=rr_buf.at[s], dst_ref=rr_buf.at[s],
                send_sem=rr_send.at[s], recv_sem=rr_recv.at[s],
                device_id=(my_x, my_y, zp if tz is None else tz),
                device_id_type=_MESH)

        def lr_desc(c, h, tz=None):
            s = 2 * c + h
            return pltpu.make_async_remote_copy(
                src_ref=lr_buf.at[s], dst_ref=lr_buf.at[s],
                send_sem=lr_send.at[s], recv_sem=lr_recv.at[s],
                device_id=(my_x, my_y, zm if tz is None else tz),
                device_id_type=_MESH)

        def rr2_desc(h):
            return pltpu.make_async_remote_copy(
                src_ref=rr2_buf.at[h], dst_ref=rr2_buf.at[h],
                send_sem=rr2_send.at[h], recv_sem=rr2_recv.at[h],
                device_id=(my_x, my_y, Z - 1), device_id_type=_MESH)

        def lr2_desc(h):
            return pltpu.make_async_remote_copy(
                src_ref=lr2_buf.at[h], dst_ref=lr2_buf.at[h],
                send_sem=lr2_send.at[h], recv_sem=lr2_recv.at[h],
                device_id=(my_x, my_y, 0), device_id_type=_MESH)

        def xh(c, h):
            return x_ref[c, pl.ds(h * H, H), :]

        @pl.when(my_z == 2)
        def _():
            for h in (0, 1):
                rr2_buf[h] = xh(3, h).astype(jnp.bfloat16)
                rr2_desc(h).start()

        @pl.when(my_z == 1)
        def _():
            for h in (0, 1):
                lr2_buf[h] = xh(0, h).astype(jnp.bfloat16)
                lr2_desc(h).start()

        def rw_edge(c, h):
            @pl.when(my_z == 0)
            def _():
                rr_buf[2 * c + h] = xh(c, h).astype(jnp.bfloat16)
                rr_desc(c, h).start()

        def rw_mid(c, h):
            cond = (my_z == 1) if c == Z - 1 else ((my_z >= 1) & (my_z < c))
            tz = (Z - 1) if c == Z - 1 else None

            @pl.when(cond)
            def _():
                rr_desc(c, h).wait_recv()
                rr_buf[2 * c + h] = (
                    rr_buf[2 * c + h] + xh(c, h).astype(jnp.bfloat16))
                rr_desc(c, h, tz).start()

        def lw_edge(c, h):
            @pl.when(my_z == Z - 1)
            def _():
                lr_buf[2 * c + h] = xh(c, h).astype(jnp.bfloat16)
                lr_desc(c, h).start()

        def lw_mid(c, h):
            cond = (my_z == 2) if c == 0 else ((my_z <= Z - 2) & (my_z > c))
            tz = 0 if c == 0 else None

            @pl.when(cond)
            def _():
                lr_desc(c, h).wait_recv()
                lr_buf[2 * c + h] = (
                    lr_buf[2 * c + h] + xh(c, h).astype(jnp.bfloat16))
                lr_desc(c, h, tz).start()

        def rw_step(c, h):
            rw_edge(c, h)
            rw_mid(c, h)

        def lw_step(c, h):
            lw_edge(c, h)
            lw_mid(c, h)

        @pl.when(my_z % 2 == 1)
        def _():
            rw_step(3, 0); lw_step(0, 0); rw_step(2, 0); lw_step(1, 0)
            rw_step(1, 0); lw_step(2, 0)

        @pl.when(my_z % 2 == 0)
        def _():
            lw_step(0, 0); rw_step(3, 0); lw_step(1, 0); rw_step(2, 0)
            lw_step(2, 0); rw_step(1, 0)

        for c in (3, 2, 1):
            rw_edge(c, 1)
        for c in (0, 1, 2):
            lw_edge(c, 1)

        r_x = 2 * (1 - my_x) + my_y
        r_y = 2 * my_x + (1 - my_y)

        def pb_desc(i, h):
            s = 2 * i + h
            if i == 0:
                sl = out_ref.at[pl.ds(r * MS + h * H, H), :]
                return pltpu.make_async_remote_copy(
                    src_ref=sl, dst_ref=sl,
                    send_sem=pb_send.at[s], recv_sem=pb_recv.at[s],
                    device_id=(1 - my_x, my_y, my_z), device_id_type=_MESH)
            if i == 1:
                sl = out_ref.at[pl.ds(r * MS + h * H, H), :]
                return pltpu.make_async_remote_copy(
                    src_ref=sl, dst_ref=sl,
                    send_sem=pb_send.at[s], recv_sem=pb_recv.at[s],
                    device_id=(my_x, 1 - my_y, my_z), device_id_type=_MESH)
            if i == 2:
                sl = out_ref.at[pl.ds(r_y * MS + h * H, HQ), :]
                return pltpu.make_async_remote_copy(
                    src_ref=sl, dst_ref=sl,
                    send_sem=pb_send.at[s], recv_sem=pb_recv.at[s],
                    device_id=(1 - my_x, my_y, my_z), device_id_type=_MESH)
            sl = out_ref.at[pl.ds(r_x * MS + h * H + HQ, HQ), :]
            return pltpu.make_async_remote_copy(
                src_ref=sl, dst_ref=sl,
                send_sem=pb_send.at[s], recv_sem=pb_recv.at[s],
                device_id=(my_x, 1 - my_y, my_z), device_id_type=_MESH)

        def combine_norm_bstart(h):
            for c in range(Z):
                if c >= 1:
                    @pl.when(my_z == c)
                    def _(c=c):
                        rr_desc(c, h).wait_recv()
                if c <= Z - 2:
                    @pl.when(my_z == c)
                    def _(c=c):
                        lr_desc(c, h).wait_recv()
                if c == Z - 1:
                    @pl.when(my_z == c)
                    def _():
                        rr2_desc(h).wait_recv()
                if c == 0:
                    @pl.when(my_z == c)
                    def _():
                        lr2_desc(h).wait_recv()

                @pl.when(my_z == c)
                def _(c=c):
                    acc = xh(c, h)
                    if c >= 1:
                        acc = acc + rr_buf[2 * c + h].astype(jnp.float32)
                    if c <= Z - 2:
                        acc = acc + lr_buf[2 * c + h].astype(jnp.float32)
                    if c == Z - 1:
                        acc = acc + rr2_buf[h].astype(jnp.float32)
                    if c == 0:
                        acc = acc + lr2_buf[h].astype(jnp.float32)
                    inv = lax.rsqrt(
                        jnp.mean(acc * acc, axis=-1, keepdims=True) + 1e-6)
                    out_ref[pl.ds(r * MS + h * H, H), :] = (
                        acc * inv * g_ref[...]).astype(jnp.bfloat16)

            pb_desc(0, h).start()
            pb_desc(1, h).start()

        def b_mid(h):
            pb_desc(0, h).wait_recv()
            pb_desc(3, h).start()
            pb_desc(1, h).wait_recv()
            pb_desc(2, h).start()

        def b_fin(h):
            pb_desc(2, h).wait_recv()
            pb_desc(3, h).wait_recv()

        combine_norm_bstart(0)

        @pl.when(my_z % 2 == 1)
        def _():
            rw_mid(3, 1); lw_mid(0, 1); rw_mid(2, 1); lw_mid(1, 1)
            rw_mid(1, 1); lw_mid(2, 1)

        @pl.when(my_z % 2 == 0)
        def _():
            lw_mid(0, 1); rw_mid(3, 1); lw_mid(1, 1); rw_mid(2, 1)
            lw_mid(2, 1); rw_mid(1, 1)

        b_mid(0)
        combine_norm_bstart(1)
        b_fin(0)
        b_mid(1)
        b_fin(1)

        for h in (0, 1):
            for i in range(4):
                pb_desc(i, h).wait_send()
            for c in range(1, Z):
                cond = (my_z <= 1) if c == Z - 1 else (my_z < c)

                @pl.when(cond)
                def _(c=c, h=h):
                    rr_desc(c, h).wait_send()
            for c in range(Z - 1):
                cond = (my_z >= 2) if c == 0 else (my_z > c)

                @pl.when(cond)
                def _(c=c, h=h):
                    lr_desc(c, h).wait_send()

            @pl.when(my_z == 2)
            def _(h=h):
                rr2_desc(h).wait_send()

            @pl.when(my_z == 1)
            def _(h=h):
                lr2_desc(h).wait_send()

    return pl.pallas_call(
        body,
        out_shape=jax.ShapeDtypeStruct((M, D), jnp.bfloat16),
        in_specs=[
            pl.BlockSpec(memory_space=pltpu.VMEM),
            pl.BlockSpec(memory_space=pltpu.VMEM),
        ],
        out_specs=pl.BlockSpec(memory_space=pltpu.VMEM),
        scratch_shapes=[
            pltpu.VMEM((2 * Z, H, D), jnp.bfloat16),
            pltpu.VMEM((2 * Z, H, D), jnp.bfloat16),
            pltpu.VMEM((2, H, D), jnp.bfloat16),
            pltpu.VMEM((2, H, D), jnp.bfloat16),
            pltpu.SemaphoreType.DMA((2 * Z,)),
            pltpu.SemaphoreType.DMA((2 * Z,)),
            pltpu.SemaphoreType.DMA((2 * Z,)),
            pltpu.SemaphoreType.DMA((2 * Z,)),
            pltpu.SemaphoreType.DMA((2,)),
            pltpu.SemaphoreType.DMA((2,)),
            pltpu.SemaphoreType.DMA((2,)),
            pltpu.SemaphoreType.DMA((2,)),
            pltpu.SemaphoreType.DMA((8,)),
            pltpu.SemaphoreType.DMA((8,)),
        ],
        compiler_params=pltpu.CompilerParams(collective_id=0),
    )(xg, gamma.reshape(1, D))
